# Initial kernel scaffold; baseline (speedup 1.0000x reference)
import jax
import jax.numpy as jnp
from jax.experimental import pallas as pl


def kernel(feat0, feat1, feat2, fc0_w, fc0_b, fc1_w, fc1_b, fc2_w, fc2_b, g0_fc, g0_eemb, g0_fce, g0_al, g0_ar, g0_ae, g1_fc, g1_eemb, g1_fce, g1_al, g1_ar, g1_ae, g2_fc, g2_eemb, g2_fce, g2_al, g2_ar, g2_ae, dec_W, edge_index, e_feat, left, right, mid):
    raise NotImplementedError("write your pallas kernel here")


# speedup vs baseline: 1.0050x; 1.0050x over previous
import functools

import jax
import jax.numpy as jnp
from jax.experimental import pallas as pl

N = 10000
E = 160000
NE = 5
ED = 64
NH = 64
NC = 64
H = 8
P = 8192
IN = 256
ALPHA = 0.05


def _round_up(x, m):
    return (x + m - 1) // m * m


def _mm(x, w, bm=512):
    M, K = x.shape
    _, Nn = w.shape
    Mp = _round_up(M, bm)
    if Mp != M:
        x = jnp.pad(x, ((0, Mp - M), (0, 0)))

    def body(xr, wr, outr):
        outr[...] = jnp.dot(xr[...], wr[...], preferred_element_type=jnp.float32)

    out = pl.pallas_call(
        body,
        grid=(Mp // bm,),
        in_specs=[
            pl.BlockSpec((bm, K), lambda i: (i, 0)),
            pl.BlockSpec((K, Nn), lambda i: (0, 0)),
        ],
        out_specs=pl.BlockSpec((bm, Nn), lambda i: (i, 0)),
        out_shape=jax.ShapeDtypeStruct((Mp, Nn), jnp.float32),
    )(x, w)
    return out[:M]


def _l2n(x):
    return x / jnp.maximum(jnp.linalg.norm(x, axis=1, keepdims=True), 1e-12)


def _edge_softmax(e, dst, n):
    m = jax.ops.segment_max(e, dst, num_segments=n)
    ex = jnp.exp(e - m[dst])
    s = jax.ops.segment_sum(ex, dst, num_segments=n)
    return ex / (s[dst] + 1e-9)


def _gat(h, fcW, eemb, fceW, al, ar, ae, src, dst, ef, res_attn, residual, act):
    d = fcW.shape[1] // H
    feat = _mm(h, fcW).reshape(-1, H, d)
    ea_rel = ((eemb @ fceW).reshape(NE, H, ED) * ae[None]).sum(-1)
    ea = ea_rel[ef]
    el = (feat * al[None]).sum(-1)
    er = (feat * ar[None]).sum(-1)
    logits = jax.nn.leaky_relu(el[src] + er[dst] + ea, 0.2)
    a = _edge_softmax(logits, dst, h.shape[0])
    if res_attn is not None:
        a = a * (1.0 - ALPHA) + res_attn * ALPHA
    rst = jax.ops.segment_sum(feat[src] * a[:, :, None], dst, num_segments=h.shape[0])
    if residual:
        rst = rst + h.reshape(h.shape[0], H, d)
    if act:
        rst = jax.nn.elu(rst)
    return rst, a


def kernel(feat0, feat1, feat2, fc0_w, fc0_b, fc1_w, fc1_b, fc2_w, fc2_b,
           g0_fc, g0_eemb, g0_fce, g0_al, g0_ar, g0_ae,
           g1_fc, g1_eemb, g1_fce, g1_al, g1_ar, g1_ae,
           g2_fc, g2_eemb, g2_fce, g2_al, g2_ar, g2_ae, dec_W,
           edge_index, e_feat, left, right, mid):
    src, dst = edge_index[0], edge_index[1]
    h = jnp.concatenate([
        _mm(feat0, fc0_w) + fc0_b,
        _mm(feat1, fc1_w) + fc1_b,
        _mm(feat2, fc2_w) + fc2_b,
    ], axis=0)
    emb = [_l2n(h)]
    h1, a1 = _gat(h, g0_fc, g0_eemb, g0_fce, g0_al, g0_ar, g0_ae, src, dst,
                  e_feat, None, False, True)
    emb.append(_l2n(h1.mean(1)))
    h1f = h1.reshape(N, -1)
    h2, a2 = _gat(h1f, g1_fc, g1_eemb, g1_fce, g1_al, g1_ar, g1_ae, src, dst,
                  e_feat, a1, True, True)
    emb.append(_l2n(h2.mean(1)))
    h2f = h2.reshape(N, -1)
    h3, _ = _gat(h2f, g2_fc, g2_eemb, g2_fce, g2_al, g2_ar, g2_ae, src, dst,
                 e_feat, a2, True, False)
    logits = _l2n(h3.mean(1))
    emb.append(logits)
    z = jnp.concatenate(emb, axis=1)
    le = z[left]
    re = z[right]
    scores = jnp.zeros((P,), jnp.float32)
    for r in range(NE):
        t = (_mm(le, dec_W[r]) * re).sum(1)
        scores = jnp.where(mid == r, t, scores)
    return jax.nn.sigmoid(scores)


# speedup vs baseline: 3.9248x; 3.9051x over previous
import functools

import jax
import jax.numpy as jnp
from jax import lax
from jax.experimental import pallas as pl
from jax.experimental.pallas import tpu as pltpu
from jax.experimental.pallas import tpu_sc as plsc

N = 10000
E = 160000
NE = 5
ED = 64
NH = 64
NC = 64
H = 8
P = 8192
IN = 256
ALPHA = 0.05

NW = 32
CH = 128
CHA = 32
EP = 163840
NCH = EP // (NW * CH)
NP = 10240
DTILE = NP // 16
DSROWS = NP // 32
_USE_SC_AGG = True
_USE_SC_PAIR = True
_USE_SC_EX = True
_USE_SC_NORM = True
NACC = 10112
ATILE = NACC // 16


def _mm(x, w, bm=512):
    M, K = x.shape
    _, Nn = w.shape

    def body(xr, wr, outr):
        outr[...] = jnp.dot(xr[...], wr[...], preferred_element_type=jnp.float32)

    return pl.pallas_call(
        body,
        grid=(M // bm,),
        in_specs=[
            pl.BlockSpec((bm, K), lambda i: (i, 0)),
            pl.BlockSpec((K, Nn), lambda i: (0, 0)),
        ],
        out_specs=pl.BlockSpec((bm, Nn), lambda i: (i, 0)),
        out_shape=jax.ShapeDtypeStruct((M, Nn), jnp.float32),
    )(x, w)


def _sc_mesh():
    return plsc.VectorSubcoreMesh(core_axis_name="c", subcore_axis_name="s")


_IOTA = None


def _iota16():
    return lax.iota(jnp.int32, 16)


def _sc_logits_part(el_flat, ea_flat, srcp, efp):
    @functools.partial(
        pl.kernel,
        mesh=_sc_mesh(),
        compiler_params=pltpu.CompilerParams(needs_layout_passes=False),
        out_type=jax.ShapeDtypeStruct((EP, 16), jnp.float32),
        scratch_types=[
            pltpu.VMEM((NP * 8,), jnp.float32),
            pltpu.VMEM((64,), jnp.float32),
            pltpu.VMEM((CH,), jnp.int32),
            pltpu.VMEM((CH,), jnp.int32),
            pltpu.VMEM((CH, 16), jnp.float32),
        ],
    )
    def k(el_h, ea_h, src_h, ef_h, part_h, el_v, ea_v, si_v, fi_v, pbuf):
        cid = lax.axis_index("c")
        tid = lax.axis_index("s")
        wid = tid * 2 + cid
        pltpu.sync_copy(el_h, el_v)
        pltpu.sync_copy(ea_h, ea_v)
        iota = _iota16()

        def chunk(j, carry):
            base = wid * (NCH * CH) + j * CH
            pltpu.sync_copy(src_h.at[pl.ds(base, CH)], si_v)
            pltpu.sync_copy(ef_h.at[pl.ds(base, CH)], fi_v)
            for g in range(CH // 16):
                s16 = si_v[pl.ds(g * 16, 16)] * 8
                f16 = fi_v[pl.ds(g * 16, 16)] * 8
                e16 = iota + (g * 16)
                for h in range(8):
                    v = plsc.load_gather(el_v, [s16 + h]) + \
                        plsc.load_gather(ea_v, [f16 + h])
                    plsc.store_scatter(pbuf, [e16, jnp.full((16,), h, jnp.int32)], v)
            pltpu.sync_copy(pbuf, part_h.at[pl.ds(base, CH)])
            return carry

        lax.fori_loop(0, NCH, chunk, 0)

    return k(el_flat, ea_flat, srcp, efp)


def _sc_logits_ex(er_flat, part, dstp):
    @functools.partial(
        pl.kernel,
        mesh=_sc_mesh(),
        compiler_params=pltpu.CompilerParams(needs_layout_passes=False),
        out_type=jax.ShapeDtypeStruct((EP, 16), jnp.float32),
        scratch_types=[
            pltpu.VMEM((NP * 8,), jnp.float32),
            pltpu.VMEM((CH,), jnp.int32),
            pltpu.VMEM((CH, 16), jnp.float32),
        ],
    )
    def k(er_h, part_h, dst_h, ex_h, er_v, di_v, pbuf):
        cid = lax.axis_index("c")
        tid = lax.axis_index("s")
        wid = tid * 2 + cid
        pltpu.sync_copy(er_h, er_v)
        iota = _iota16()

        def chunk(j, carry):
            base = wid * (NCH * CH) + j * CH
            pltpu.sync_copy(dst_h.at[pl.ds(base, CH)], di_v)
            pltpu.sync_copy(part_h.at[pl.ds(base, CH)], pbuf)
            for g in range(CH // 16):
                d16 = di_v[pl.ds(g * 16, 16)] * 8
                e16 = iota + (g * 16)
                for h in range(8):
                    h16 = jnp.full((16,), h, jnp.int32)
                    x = plsc.load_gather(pbuf, [e16, h16]) + \
                        plsc.load_gather(er_v, [d16 + h])
                    x = jnp.where(x > 0.0, x, x * 0.2)
                    v = jnp.exp(jnp.minimum(x, 60.0))
                    plsc.store_scatter(pbuf, [e16, h16], v)
            pltpu.sync_copy(pbuf, ex_h.at[pl.ds(base, CH)])
            return carry

        lax.fori_loop(0, NCH, chunk, 0)

    return k(er_flat, part, dstp)


def _sc_den_scatter(ex, dstp, zden):
    @functools.partial(
        pl.kernel,
        mesh=_sc_mesh(),
        compiler_params=pltpu.CompilerParams(needs_layout_passes=False),
        out_type=jax.ShapeDtypeStruct((2, NP, 16), jnp.float32),
        scratch_types=[
            pltpu.VMEM((CH,), jnp.int32),
            pltpu.VMEM((CH, 16), jnp.float32),
            pltpu.VMEM_SHARED((NP, 16), jnp.float32),
        ],
    )
    def k(ex_h, dst_h, zden_h, den_h, di_v, exb, den_sp):
        cid = lax.axis_index("c")
        tid = lax.axis_index("s")
        wid = tid * 2 + cid
        pltpu.sync_copy(zden_h.at[pl.ds(tid * DTILE, DTILE)],
                        den_sp.at[pl.ds(tid * DTILE, DTILE)])
        plsc.subcore_barrier()

        def chunk(j, carry):
            base = wid * (NCH * CH) + j * CH
            pltpu.sync_copy(dst_h.at[pl.ds(base, CH)], di_v)
            pltpu.sync_copy(ex_h.at[pl.ds(base, CH)], exb)
            pltpu.sync_copy(exb, den_sp.at[di_v], add=True)
            return carry

        lax.fori_loop(0, NCH, chunk, 0)
        plsc.subcore_barrier()
        pltpu.sync_copy(den_sp.at[pl.ds(tid * DTILE, DTILE)],
                        den_h.at[cid, pl.ds(tid * DTILE, DTILE)])

    return k(ex, dstp, zden)


def _sc_den_sum(den):
    @functools.partial(
        pl.kernel,
        mesh=_sc_mesh(),
        compiler_params=pltpu.CompilerParams(needs_layout_passes=False),
        out_type=jax.ShapeDtypeStruct((NP * 8,), jnp.float32),
        scratch_types=[
            pltpu.VMEM((DSROWS, 16), jnp.float32),
            pltpu.VMEM((DSROWS, 16), jnp.float32),
            pltpu.VMEM((DSROWS * 8,), jnp.float32),
        ],
    )
    def k(den_h, out_h, d0, d1, dc):
        cid = lax.axis_index("c")
        tid = lax.axis_index("s")
        wid = tid * 2 + cid
        r0 = wid * DSROWS
        pltpu.sync_copy(den_h.at[0, pl.ds(r0, DSROWS)], d0)
        pltpu.sync_copy(den_h.at[1, pl.ds(r0, DSROWS)], d1)
        iota = _iota16()
        rows = iota // 8
        cols = iota % 8

        def pair(j, carry):
            ri = rows + j * 2
            v = plsc.load_gather(d0, [ri, cols]) + plsc.load_gather(d1, [ri, cols])
            dc[pl.ds(j * 16, 16)] = v
            return carry

        lax.fori_loop(0, DSROWS // 2, pair, 0)
        pltpu.sync_copy(dc, out_h.at[pl.ds(wid * DSROWS * 8, DSROWS * 8)])

    return k(den)


def _sc_norm(ex, den_sum, dstp, res):
    has_res = res is not None

    @functools.partial(
        pl.kernel,
        mesh=_sc_mesh(),
        compiler_params=pltpu.CompilerParams(needs_layout_passes=False),
        out_type=jax.ShapeDtypeStruct((EP, 16), jnp.float32),
        scratch_types=[
            pltpu.VMEM((NP * 8,), jnp.float32),
            pltpu.VMEM((CH,), jnp.int32),
            pltpu.VMEM((CH, 16), jnp.float32),
            pltpu.VMEM((CH, 16), jnp.float32),
        ],
    )
    def k(ex_h, den_h, dst_h, *args):
        if has_res:
            (res_h, a_h, den_v, di_v, exb, rb) = args
        else:
            (a_h, den_v, di_v, exb, rb) = args
        cid = lax.axis_index("c")
        tid = lax.axis_index("s")
        wid = tid * 2 + cid
        pltpu.sync_copy(den_h, den_v)
        iota = _iota16()

        def chunk_a(j, carry):
            base = wid * (NCH * CH) + j * CH
            pltpu.sync_copy(dst_h.at[pl.ds(base, CH)], di_v)
            pltpu.sync_copy(ex_h.at[pl.ds(base, CH)], exb)
            if has_res:
                pltpu.sync_copy(res_h.at[pl.ds(base, CH)], rb)
            for g in range(CH // 16):
                d16 = di_v[pl.ds(g * 16, 16)] * 8
                e16 = iota + (g * 16)
                for h in range(8):
                    h16 = jnp.full((16,), h, jnp.int32)
                    num = plsc.load_gather(exb, [e16, h16])
                    dd = plsc.load_gather(den_v, [d16 + h])
                    a = num / (dd + 1e-9)
                    if has_res:
                        a = a * (1.0 - ALPHA) + \
                            plsc.load_gather(rb, [e16, h16]) * ALPHA
                    plsc.store_scatter(exb, [e16, h16], a)
            pltpu.sync_copy(exb, a_h.at[pl.ds(base, CH)])
            return carry

        lax.fori_loop(0, NCH, chunk_a, 0)

    if has_res:
        return k(ex, den_sum, dstp, res)
    return k(ex, den_sum, dstp)


def _sc_gather_scale(a, srcp, featT2):
    @functools.partial(
        pl.kernel,
        mesh=_sc_mesh(),
        compiler_params=pltpu.CompilerParams(needs_layout_passes=False),
        out_type=jax.ShapeDtypeStruct((4, EP, 128), jnp.float32),
        scratch_types=[
            pltpu.VMEM((CHA,), jnp.int32),
            pltpu.VMEM((CHA,), jnp.int32),
            pltpu.VMEM((CHA, 16), jnp.float32),
            pltpu.VMEM((CHA, 128), jnp.float32),
            pltpu.VMEM((CHA, 128), jnp.float32),
        ],
    )
    def k(a_h, src_h, ft_h, msg_h, si_v, gi_v, ab, fbuf, msg):
        cid = lax.axis_index("c")
        tid = lax.axis_index("s")
        wid = tid * 2 + cid

        for hg in range(4):
            def chunk_g(j, carry):
                base = wid * (NCH * CH) + j * CHA
                pltpu.sync_copy(src_h.at[pl.ds(base, CHA)], si_v)
                pltpu.sync_copy(a_h.at[pl.ds(base, CHA)], ab)
                for g in range(CHA // 16):
                    gi_v[pl.ds(g * 16, 16)] = si_v[pl.ds(g * 16, 16)] * 4 + hg
                pltpu.sync_copy(ft_h.at[gi_v], fbuf)
                h0 = jnp.full((16,), 2 * hg, jnp.int32)
                h1 = h0 + 1

                def edge(e, carry2):
                    e16 = jnp.full((16,), e, jnp.int32)
                    av0 = plsc.load_gather(ab, [e16, h0])
                    av1 = plsc.load_gather(ab, [e16, h1])
                    for q in range(4):
                        msg[e, pl.ds(q * 16, 16)] = \
                            fbuf[e, pl.ds(q * 16, 16)] * av0
                        msg[e, pl.ds(64 + q * 16, 16)] = \
                            fbuf[e, pl.ds(64 + q * 16, 16)] * av1
                    return carry2

                lax.fori_loop(0, CHA, edge, 0)
                pltpu.sync_copy(msg, msg_h.at[hg, pl.ds(base, CHA)])
                return carry

            lax.fori_loop(0, NCH * 4, chunk_g, 0)

    return k(a, srcp, featT2)


def _sc_pair_gather(z, left, right):
    CP = 128

    @functools.partial(
        pl.kernel,
        mesh=_sc_mesh(),
        compiler_params=pltpu.CompilerParams(needs_layout_passes=False),
        out_type=(
            jax.ShapeDtypeStruct((P, 256), jnp.float32),
            jax.ShapeDtypeStruct((P, 256), jnp.float32),
        ),
        scratch_types=[
            pltpu.VMEM((CP,), jnp.int32),
            pltpu.VMEM((CP,), jnp.int32),
            pltpu.VMEM((CP, 256), jnp.float32),
            pltpu.VMEM((CP, 256), jnp.float32),
        ],
    )
    def k(z_h, l_h, r_h, le_h, re_h, li_v, ri_v, lrow, rrow):
        cid = lax.axis_index("c")
        tid = lax.axis_index("s")
        wid = tid * 2 + cid

        def step(j, carry):
            base = wid * (P // NW) + j * CP
            pltpu.sync_copy(l_h.at[pl.ds(base, CP)], li_v)
            pltpu.sync_copy(r_h.at[pl.ds(base, CP)], ri_v)
            pltpu.sync_copy(z_h.at[li_v], lrow)
            pltpu.sync_copy(z_h.at[ri_v], rrow)
            pltpu.sync_copy(lrow, le_h.at[pl.ds(base, CP)])
            pltpu.sync_copy(rrow, re_h.at[pl.ds(base, CP)])
            return carry

        lax.fori_loop(0, (P // NW) // CP, step, 0)

    return k(z, left, right)


def _l2n(x):
    return x / jnp.maximum(jnp.linalg.norm(x, axis=1, keepdims=True), 1e-12)


def _attn_proj(al, ar):
    d = al.shape[1]
    blocks = []
    for h in range(H):
        z = jnp.zeros((d, 16), jnp.float32)
        z = z.at[:, h].set(al[h]).at[:, 8 + h].set(ar[h])
        blocks.append(z)
    return jnp.concatenate(blocks, axis=0)


def _gat(h, fcW, eemb, fceW, al, ar, ae, srcp, dstp, efp, zden, zacc,
         res_attn, residual, act):
    feat = _mm(h, fcW)
    elr = _mm(feat, _attn_proj(al, ar))
    ea_rel = ((eemb @ fceW).reshape(NE, H, ED) * ae[None]).sum(-1)
    el_flat = elr[:, :8].reshape(NP * 8)
    er_flat = elr[:, 8:].reshape(NP * 8)
    ea_flat = jnp.zeros((8, 8), jnp.float32).at[:NE].set(ea_rel).reshape(64)

    part = _sc_logits_part(el_flat, ea_flat, srcp, efp)
    ex = _sc_logits_ex(er_flat, part, dstp)
    den = jax.ops.segment_sum(ex[:, :8], dstp, num_segments=NP)
    den_sum = den.reshape(NP * 8)
    featT2 = feat.reshape(NP * 4, 128)
    a = _sc_norm(ex, den_sum, dstp, res_attn)
    msg = _sc_gather_scale(a, srcp, featT2)
    accs = jax.ops.segment_sum(
        msg.reshape(4 * EP, 128),
        (jnp.arange(4, dtype=jnp.int32)[:, None] * NP + dstp[None, :]).reshape(-1),
        num_segments=4 * NP).reshape(4, NP, 128)
    rst = jnp.transpose(accs, (1, 0, 2)).reshape(NP, 512)
    if residual:
        rst = rst + h
    if act:
        rst = jax.nn.elu(rst)
    return rst, a


def kernel(feat0, feat1, feat2, fc0_w, fc0_b, fc1_w, fc1_b, fc2_w, fc2_b,
           g0_fc, g0_eemb, g0_fce, g0_al, g0_ar, g0_ae,
           g1_fc, g1_eemb, g1_fce, g1_al, g1_ar, g1_ae,
           g2_fc, g2_eemb, g2_fce, g2_al, g2_ar, g2_ae, dec_W,
           edge_index, e_feat, left, right, mid):
    src, dst = edge_index[0], edge_index[1]
    pad = jnp.full((EP - E,), N, jnp.int32)
    srcp = jnp.concatenate([src.astype(jnp.int32), pad])
    dstp = jnp.concatenate([dst.astype(jnp.int32), pad])
    efp = jnp.concatenate([e_feat.astype(jnp.int32),
                           jnp.zeros((EP - E,), jnp.int32)])
    zden = jnp.zeros((NP, 16), jnp.float32)
    zacc = jnp.zeros((NP, 128), jnp.float32)

    h0 = jnp.concatenate([
        _mm(jnp.pad(feat0, ((0, 96), (0, 0))), fc0_w)[:4000] + fc0_b,
        _mm(jnp.pad(feat1, ((0, 72), (0, 0))), fc1_w)[:3000] + fc1_b,
        _mm(jnp.pad(feat2, ((0, 72), (0, 0))), fc2_w)[:3000] + fc2_b,
    ], axis=0)
    h0 = jnp.pad(h0, ((0, NP - N), (0, 0)))
    emb0 = _l2n(h0[:N])

    h1, a1 = _gat(h0, g0_fc, g0_eemb, g0_fce, g0_al, g0_ar, g0_ae,
                  srcp, dstp, efp, zden, zacc, None, False, True)
    emb1 = _l2n(h1[:N].reshape(N, H, NH).mean(1))
    h2, a2 = _gat(h1, g1_fc, g1_eemb, g1_fce, g1_al, g1_ar, g1_ae,
                  srcp, dstp, efp, zden, zacc, a1, True, True)
    emb2 = _l2n(h2[:N].reshape(N, H, NH).mean(1))
    h3, _ = _gat(h2, g2_fc, g2_eemb, g2_fce, g2_al, g2_ar, g2_ae,
                 srcp, dstp, efp, zden, zacc, a2, True, False)
    emb3 = _l2n(h3[:N].reshape(N, H, NC).mean(1))

    z = jnp.concatenate([emb0, emb1, emb2, emb3], axis=1)
    if _USE_SC_PAIR:
        le, re = _sc_pair_gather(z, left.astype(jnp.int32),
                                 right.astype(jnp.int32))
    else:
        le, re = z[left], z[right]
    scores = jnp.zeros((P,), jnp.float32)
    for r in range(NE):
        t = (_mm(le, dec_W[r]) * re).sum(1)
        scores = jnp.where(mid == r, t, scores)
    return jax.nn.sigmoid(scores)


# speedup vs baseline: 4.5116x; 1.1495x over previous
import functools

import jax
import jax.numpy as jnp
from jax import lax
from jax.experimental import pallas as pl
from jax.experimental.pallas import tpu as pltpu
from jax.experimental.pallas import tpu_sc as plsc

N = 10000
E = 160000
NE = 5
ED = 64
NH = 64
NC = 64
H = 8
P = 8192
IN = 256
ALPHA = 0.05

NW = 32
CH = 128
CHA = 128
EP = 163840
NCH = EP // (NW * CH)
NP = 10240
DTILE = NP // 16
DSROWS = NP // 32
_USE_SC_AGG = True
_USE_SC_PAIR = True
_USE_SC_EX = True
_USE_SC_NORM = True
NACC = 10112
ATILE = NACC // 16


def _mm(x, w, bm=512):
    M, K = x.shape
    _, Nn = w.shape

    def body(xr, wr, outr):
        outr[...] = jnp.dot(xr[...], wr[...], preferred_element_type=jnp.float32)

    return pl.pallas_call(
        body,
        grid=(M // bm,),
        in_specs=[
            pl.BlockSpec((bm, K), lambda i: (i, 0)),
            pl.BlockSpec((K, Nn), lambda i: (0, 0)),
        ],
        out_specs=pl.BlockSpec((bm, Nn), lambda i: (i, 0)),
        out_shape=jax.ShapeDtypeStruct((M, Nn), jnp.float32),
    )(x, w)


def _sc_mesh():
    return plsc.VectorSubcoreMesh(core_axis_name="c", subcore_axis_name="s")


_IOTA = None


def _iota16():
    return lax.iota(jnp.int32, 16)


def _sc_logits_part(el_flat, ea_flat, srcp, efp):
    @functools.partial(
        pl.kernel,
        mesh=_sc_mesh(),
        compiler_params=pltpu.CompilerParams(needs_layout_passes=False),
        out_type=jax.ShapeDtypeStruct((EP, 16), jnp.float32),
        scratch_types=[
            pltpu.VMEM((NP * 8,), jnp.float32),
            pltpu.VMEM((64,), jnp.float32),
            pltpu.VMEM((CH,), jnp.int32),
            pltpu.VMEM((CH,), jnp.int32),
            pltpu.VMEM((CH, 16), jnp.float32),
        ],
    )
    def k(el_h, ea_h, src_h, ef_h, part_h, el_v, ea_v, si_v, fi_v, pbuf):
        cid = lax.axis_index("c")
        tid = lax.axis_index("s")
        wid = tid * 2 + cid
        pltpu.sync_copy(el_h, el_v)
        pltpu.sync_copy(ea_h, ea_v)
        iota = _iota16()

        def chunk(j, carry):
            base = wid * (NCH * CH) + j * CH
            pltpu.sync_copy(src_h.at[pl.ds(base, CH)], si_v)
            pltpu.sync_copy(ef_h.at[pl.ds(base, CH)], fi_v)
            for g in range(CH // 16):
                s16 = si_v[pl.ds(g * 16, 16)] * 8
                f16 = fi_v[pl.ds(g * 16, 16)] * 8
                e16 = iota + (g * 16)
                for h in range(8):
                    v = plsc.load_gather(el_v, [s16 + h]) + \
                        plsc.load_gather(ea_v, [f16 + h])
                    plsc.store_scatter(pbuf, [e16, jnp.full((16,), h, jnp.int32)], v)
            pltpu.sync_copy(pbuf, part_h.at[pl.ds(base, CH)])
            return carry

        lax.fori_loop(0, NCH, chunk, 0)

    return k(el_flat, ea_flat, srcp, efp)


def _sc_logits_ex(er_flat, part, dstp):
    @functools.partial(
        pl.kernel,
        mesh=_sc_mesh(),
        compiler_params=pltpu.CompilerParams(needs_layout_passes=False),
        out_type=jax.ShapeDtypeStruct((EP, 16), jnp.float32),
        scratch_types=[
            pltpu.VMEM((NP * 8,), jnp.float32),
            pltpu.VMEM((CH,), jnp.int32),
            pltpu.VMEM((CH, 16), jnp.float32),
        ],
    )
    def k(er_h, part_h, dst_h, ex_h, er_v, di_v, pbuf):
        cid = lax.axis_index("c")
        tid = lax.axis_index("s")
        wid = tid * 2 + cid
        pltpu.sync_copy(er_h, er_v)
        iota = _iota16()

        def chunk(j, carry):
            base = wid * (NCH * CH) + j * CH
            pltpu.sync_copy(dst_h.at[pl.ds(base, CH)], di_v)
            pltpu.sync_copy(part_h.at[pl.ds(base, CH)], pbuf)
            for g in range(CH // 16):
                d16 = di_v[pl.ds(g * 16, 16)] * 8
                e16 = iota + (g * 16)
                for h in range(8):
                    h16 = jnp.full((16,), h, jnp.int32)
                    x = plsc.load_gather(pbuf, [e16, h16]) + \
                        plsc.load_gather(er_v, [d16 + h])
                    x = jnp.where(x > 0.0, x, x * 0.2)
                    v = jnp.exp(jnp.minimum(x, 60.0))
                    plsc.store_scatter(pbuf, [e16, h16], v)
            pltpu.sync_copy(pbuf, ex_h.at[pl.ds(base, CH)])
            return carry

        lax.fori_loop(0, NCH, chunk, 0)

    return k(er_flat, part, dstp)


def _sc_norm(ex, den_sum, dstp, res):
    has_res = res is not None

    @functools.partial(
        pl.kernel,
        mesh=_sc_mesh(),
        compiler_params=pltpu.CompilerParams(needs_layout_passes=False),
        out_type=jax.ShapeDtypeStruct((EP, 16), jnp.float32),
        scratch_types=[
            pltpu.VMEM((NP * 8,), jnp.float32),
            pltpu.VMEM((CH,), jnp.int32),
            pltpu.VMEM((CH, 16), jnp.float32),
            pltpu.VMEM((CH, 16), jnp.float32),
        ],
    )
    def k(ex_h, den_h, dst_h, *args):
        if has_res:
            (res_h, a_h, den_v, di_v, exb, rb) = args
        else:
            (a_h, den_v, di_v, exb, rb) = args
        cid = lax.axis_index("c")
        tid = lax.axis_index("s")
        wid = tid * 2 + cid
        pltpu.sync_copy(den_h, den_v)
        iota = _iota16()

        def chunk_a(j, carry):
            base = wid * (NCH * CH) + j * CH
            pltpu.sync_copy(dst_h.at[pl.ds(base, CH)], di_v)
            pltpu.sync_copy(ex_h.at[pl.ds(base, CH)], exb)
            if has_res:
                pltpu.sync_copy(res_h.at[pl.ds(base, CH)], rb)
            for g in range(CH // 16):
                d16 = di_v[pl.ds(g * 16, 16)] * 8
                e16 = iota + (g * 16)
                for h in range(8):
                    h16 = jnp.full((16,), h, jnp.int32)
                    num = plsc.load_gather(exb, [e16, h16])
                    dd = plsc.load_gather(den_v, [d16 + h])
                    a = num / (dd + 1e-9)
                    if has_res:
                        a = a * (1.0 - ALPHA) + \
                            plsc.load_gather(rb, [e16, h16]) * ALPHA
                    plsc.store_scatter(exb, [e16, h16], a)
            pltpu.sync_copy(exb, a_h.at[pl.ds(base, CH)])
            return carry

        lax.fori_loop(0, NCH, chunk_a, 0)

    if has_res:
        return k(ex, den_sum, dstp, res)
    return k(ex, den_sum, dstp)


def _sc_gather_scale(a, srcp, featT2):
    @functools.partial(
        pl.kernel,
        mesh=_sc_mesh(),
        compiler_params=pltpu.CompilerParams(needs_layout_passes=False),
        out_type=jax.ShapeDtypeStruct((4, EP, 128), jnp.float32),
        scratch_types=[
            pltpu.VMEM((CHA,), jnp.int32),
            pltpu.VMEM((CHA,), jnp.int32),
            pltpu.VMEM((CHA, 16), jnp.float32),
            pltpu.VMEM((CHA, 128), jnp.float32),
            pltpu.VMEM((CHA, 128), jnp.float32),
        ],
    )
    def k(a_h, src_h, ft_h, msg_h, si_v, gi_v, ab, fbuf, msg):
        cid = lax.axis_index("c")
        tid = lax.axis_index("s")
        wid = tid * 2 + cid

        for hg in range(4):
            def chunk_g(j, carry):
                base = wid * (NCH * CH) + j * CHA
                pltpu.sync_copy(src_h.at[pl.ds(base, CHA)], si_v)
                pltpu.sync_copy(a_h.at[pl.ds(base, CHA)], ab)
                for g in range(CHA // 16):
                    gi_v[pl.ds(g * 16, 16)] = si_v[pl.ds(g * 16, 16)] * 4 + hg
                pltpu.sync_copy(ft_h.at[gi_v], fbuf)
                h0 = jnp.full((16,), 2 * hg, jnp.int32)
                h1 = h0 + 1

                def edge(e, carry2):
                    e16 = jnp.full((16,), e, jnp.int32)
                    av0 = plsc.load_gather(ab, [e16, h0])
                    av1 = plsc.load_gather(ab, [e16, h1])
                    for q in range(4):
                        msg[e, pl.ds(q * 16, 16)] = \
                            fbuf[e, pl.ds(q * 16, 16)] * av0
                        msg[e, pl.ds(64 + q * 16, 16)] = \
                            fbuf[e, pl.ds(64 + q * 16, 16)] * av1
                    return carry2

                lax.fori_loop(0, CHA, edge, 0)
                pltpu.sync_copy(msg, msg_h.at[hg, pl.ds(base, CHA)])
                return carry

            lax.fori_loop(0, NCH, chunk_g, 0)

    return k(a, srcp, featT2)


def _sc_pair_gather(z, left, right):
    CP = 128

    @functools.partial(
        pl.kernel,
        mesh=_sc_mesh(),
        compiler_params=pltpu.CompilerParams(needs_layout_passes=False),
        out_type=(
            jax.ShapeDtypeStruct((P, 256), jnp.float32),
            jax.ShapeDtypeStruct((P, 256), jnp.float32),
        ),
        scratch_types=[
            pltpu.VMEM((CP,), jnp.int32),
            pltpu.VMEM((CP,), jnp.int32),
            pltpu.VMEM((CP, 256), jnp.float32),
            pltpu.VMEM((CP, 256), jnp.float32),
        ],
    )
    def k(z_h, l_h, r_h, le_h, re_h, li_v, ri_v, lrow, rrow):
        cid = lax.axis_index("c")
        tid = lax.axis_index("s")
        wid = tid * 2 + cid

        def step(j, carry):
            base = wid * (P // NW) + j * CP
            pltpu.sync_copy(l_h.at[pl.ds(base, CP)], li_v)
            pltpu.sync_copy(r_h.at[pl.ds(base, CP)], ri_v)
            pltpu.sync_copy(z_h.at[li_v], lrow)
            pltpu.sync_copy(z_h.at[ri_v], rrow)
            pltpu.sync_copy(lrow, le_h.at[pl.ds(base, CP)])
            pltpu.sync_copy(rrow, re_h.at[pl.ds(base, CP)])
            return carry

        lax.fori_loop(0, (P // NW) // CP, step, 0)

    return k(z, left, right)


def _l2n(x):
    return x / jnp.maximum(jnp.linalg.norm(x, axis=1, keepdims=True), 1e-12)


def _attn_proj(al, ar):
    d = al.shape[1]
    blocks = []
    for h in range(H):
        z = jnp.zeros((d, 16), jnp.float32)
        z = z.at[:, h].set(al[h]).at[:, 8 + h].set(ar[h])
        blocks.append(z)
    return jnp.concatenate(blocks, axis=0)


def _gat(h, fcW, eemb, fceW, al, ar, ae, srcp, dstp, efp,
         res_attn, residual, act):
    feat = _mm(h, fcW)
    elr = _mm(feat, _attn_proj(al, ar))
    ea_rel = ((eemb @ fceW).reshape(NE, H, ED) * ae[None]).sum(-1)
    el_flat = elr[:, :8].reshape(NP * 8)
    er_flat = elr[:, 8:].reshape(NP * 8)
    ea_flat = jnp.zeros((8, 8), jnp.float32).at[:NE].set(ea_rel).reshape(64)

    part = _sc_logits_part(el_flat, ea_flat, srcp, efp)
    ex = _sc_logits_ex(er_flat, part, dstp)
    den = jax.ops.segment_sum(ex[:, :8], dstp, num_segments=NP)
    den_sum = den.reshape(NP * 8)
    featT2 = feat.reshape(NP * 4, 128)
    a = _sc_norm(ex, den_sum, dstp, res_attn)
    msg = _sc_gather_scale(a, srcp, featT2)
    accs = jax.ops.segment_sum(
        msg.reshape(4 * EP, 128),
        (jnp.arange(4, dtype=jnp.int32)[:, None] * NP + dstp[None, :]).reshape(-1),
        num_segments=4 * NP).reshape(4, NP, 128)
    rst = jnp.transpose(accs, (1, 0, 2)).reshape(NP, 512)
    if residual:
        rst = rst + h
    if act:
        rst = jax.nn.elu(rst)
    return rst, a


def kernel(feat0, feat1, feat2, fc0_w, fc0_b, fc1_w, fc1_b, fc2_w, fc2_b,
           g0_fc, g0_eemb, g0_fce, g0_al, g0_ar, g0_ae,
           g1_fc, g1_eemb, g1_fce, g1_al, g1_ar, g1_ae,
           g2_fc, g2_eemb, g2_fce, g2_al, g2_ar, g2_ae, dec_W,
           edge_index, e_feat, left, right, mid):
    src, dst = edge_index[0], edge_index[1]
    pad = jnp.full((EP - E,), N, jnp.int32)
    srcp = jnp.concatenate([src.astype(jnp.int32), pad])
    dstp = jnp.concatenate([dst.astype(jnp.int32), pad])
    efp = jnp.concatenate([e_feat.astype(jnp.int32),
                           jnp.zeros((EP - E,), jnp.int32)])

    h0 = jnp.concatenate([
        _mm(jnp.pad(feat0, ((0, 96), (0, 0))), fc0_w)[:4000] + fc0_b,
        _mm(jnp.pad(feat1, ((0, 72), (0, 0))), fc1_w)[:3000] + fc1_b,
        _mm(jnp.pad(feat2, ((0, 72), (0, 0))), fc2_w)[:3000] + fc2_b,
    ], axis=0)
    h0 = jnp.pad(h0, ((0, NP - N), (0, 0)))
    emb0 = _l2n(h0[:N])

    h1, a1 = _gat(h0, g0_fc, g0_eemb, g0_fce, g0_al, g0_ar, g0_ae,
                  srcp, dstp, efp, None, False, True)
    emb1 = _l2n(h1[:N].reshape(N, H, NH).mean(1))
    h2, a2 = _gat(h1, g1_fc, g1_eemb, g1_fce, g1_al, g1_ar, g1_ae,
                  srcp, dstp, efp, a1, True, True)
    emb2 = _l2n(h2[:N].reshape(N, H, NH).mean(1))
    h3, _ = _gat(h2, g2_fc, g2_eemb, g2_fce, g2_al, g2_ar, g2_ae,
                 srcp, dstp, efp, a2, True, False)
    emb3 = _l2n(h3[:N].reshape(N, H, NC).mean(1))

    z = jnp.concatenate([emb0, emb1, emb2, emb3], axis=1)
    if _USE_SC_PAIR:
        le, re = _sc_pair_gather(z, left.astype(jnp.int32),
                                 right.astype(jnp.int32))
    else:
        le, re = z[left], z[right]
    scores = jnp.zeros((P,), jnp.float32)
    for r in range(NE):
        t = (_mm(le, dec_W[r]) * re).sum(1)
        scores = jnp.where(mid == r, t, scores)
    return jax.nn.sigmoid(scores)


# speedup vs baseline: 4.5129x; 1.0003x over previous
import functools

import jax
import jax.numpy as jnp
from jax import lax
from jax.experimental import pallas as pl
from jax.experimental.pallas import tpu as pltpu
from jax.experimental.pallas import tpu_sc as plsc

N = 10000
E = 160000
NE = 5
ED = 64
NH = 64
NC = 64
H = 8
P = 8192
IN = 256
ALPHA = 0.05

NW = 32
CH = 128
CHA = 128
EP = 163840
NCH = EP // (NW * CH)
NP = 10240
DTILE = NP // 16
DSROWS = NP // 32
_USE_SC_AGG = True
_USE_SC_PAIR = True
_USE_SC_EX = True
_USE_SC_NORM = True
NACC = 10112
ATILE = NACC // 16


def _mm(x, w, bm=512):
    M, K = x.shape
    _, Nn = w.shape

    def body(xr, wr, outr):
        outr[...] = jnp.dot(xr[...], wr[...], preferred_element_type=jnp.float32)

    return pl.pallas_call(
        body,
        grid=(M // bm,),
        in_specs=[
            pl.BlockSpec((bm, K), lambda i: (i, 0)),
            pl.BlockSpec((K, Nn), lambda i: (0, 0)),
        ],
        out_specs=pl.BlockSpec((bm, Nn), lambda i: (i, 0)),
        out_shape=jax.ShapeDtypeStruct((M, Nn), jnp.float32),
    )(x, w)


def _sc_mesh():
    return plsc.VectorSubcoreMesh(core_axis_name="c", subcore_axis_name="s")


_IOTA = None


def _iota16():
    return lax.iota(jnp.int32, 16)


def _sc_logits_part(el_flat, ea_flat, srcp, efp):
    @functools.partial(
        pl.kernel,
        mesh=_sc_mesh(),
        compiler_params=pltpu.CompilerParams(needs_layout_passes=False),
        out_type=jax.ShapeDtypeStruct((EP, 16), jnp.float32),
        scratch_types=[
            pltpu.VMEM((NP * 8,), jnp.float32),
            pltpu.VMEM((64,), jnp.float32),
            pltpu.VMEM((CH,), jnp.int32),
            pltpu.VMEM((CH,), jnp.int32),
            pltpu.VMEM((CH, 16), jnp.float32),
        ],
    )
    def k(el_h, ea_h, src_h, ef_h, part_h, el_v, ea_v, si_v, fi_v, pbuf):
        cid = lax.axis_index("c")
        tid = lax.axis_index("s")
        wid = tid * 2 + cid
        pltpu.sync_copy(el_h, el_v)
        pltpu.sync_copy(ea_h, ea_v)
        iota = _iota16()

        def chunk(j, carry):
            base = wid * (NCH * CH) + j * CH
            pltpu.sync_copy(src_h.at[pl.ds(base, CH)], si_v)
            pltpu.sync_copy(ef_h.at[pl.ds(base, CH)], fi_v)
            for g in range(CH // 16):
                s16 = si_v[pl.ds(g * 16, 16)] * 8
                f16 = fi_v[pl.ds(g * 16, 16)] * 8
                e16 = iota + (g * 16)
                for h in range(8):
                    v = plsc.load_gather(el_v, [s16 + h]) + \
                        plsc.load_gather(ea_v, [f16 + h])
                    plsc.store_scatter(pbuf, [e16, jnp.full((16,), h, jnp.int32)], v)
            pltpu.sync_copy(pbuf, part_h.at[pl.ds(base, CH)])
            return carry

        lax.fori_loop(0, NCH, chunk, 0)

    return k(el_flat, ea_flat, srcp, efp)


def _sc_logits_ex(er_flat, part, dstp):
    @functools.partial(
        pl.kernel,
        mesh=_sc_mesh(),
        compiler_params=pltpu.CompilerParams(needs_layout_passes=False),
        out_type=jax.ShapeDtypeStruct((EP, 16), jnp.float32),
        scratch_types=[
            pltpu.VMEM((NP * 8,), jnp.float32),
            pltpu.VMEM((CH,), jnp.int32),
            pltpu.VMEM((CH, 16), jnp.float32),
        ],
    )
    def k(er_h, part_h, dst_h, ex_h, er_v, di_v, pbuf):
        cid = lax.axis_index("c")
        tid = lax.axis_index("s")
        wid = tid * 2 + cid
        pltpu.sync_copy(er_h, er_v)
        iota = _iota16()

        def chunk(j, carry):
            base = wid * (NCH * CH) + j * CH
            pltpu.sync_copy(dst_h.at[pl.ds(base, CH)], di_v)
            pltpu.sync_copy(part_h.at[pl.ds(base, CH)], pbuf)
            for g in range(CH // 16):
                d16 = di_v[pl.ds(g * 16, 16)] * 8
                e16 = iota + (g * 16)
                for h in range(8):
                    h16 = jnp.full((16,), h, jnp.int32)
                    x = plsc.load_gather(pbuf, [e16, h16]) + \
                        plsc.load_gather(er_v, [d16 + h])
                    x = jnp.where(x > 0.0, x, x * 0.2)
                    v = jnp.exp(jnp.minimum(x, 60.0))
                    plsc.store_scatter(pbuf, [e16, h16], v)
            pltpu.sync_copy(pbuf, ex_h.at[pl.ds(base, CH)])
            return carry

        lax.fori_loop(0, NCH, chunk, 0)

    return k(er_flat, part, dstp)


def _sc_norm(ex, den_sum, dstp, res):
    has_res = res is not None

    @functools.partial(
        pl.kernel,
        mesh=_sc_mesh(),
        compiler_params=pltpu.CompilerParams(needs_layout_passes=False),
        out_type=jax.ShapeDtypeStruct((EP, 16), jnp.float32),
        scratch_types=[
            pltpu.VMEM((NP * 8,), jnp.float32),
            pltpu.VMEM((CH,), jnp.int32),
            pltpu.VMEM((CH, 16), jnp.float32),
            pltpu.VMEM((CH, 16), jnp.float32),
        ],
    )
    def k(ex_h, den_h, dst_h, *args):
        if has_res:
            (res_h, a_h, den_v, di_v, exb, rb) = args
        else:
            (a_h, den_v, di_v, exb, rb) = args
        cid = lax.axis_index("c")
        tid = lax.axis_index("s")
        wid = tid * 2 + cid
        pltpu.sync_copy(den_h, den_v)
        iota = _iota16()

        def chunk_a(j, carry):
            base = wid * (NCH * CH) + j * CH
            pltpu.sync_copy(dst_h.at[pl.ds(base, CH)], di_v)
            pltpu.sync_copy(ex_h.at[pl.ds(base, CH)], exb)
            if has_res:
                pltpu.sync_copy(res_h.at[pl.ds(base, CH)], rb)
            for g in range(CH // 16):
                d16 = di_v[pl.ds(g * 16, 16)] * 8
                e16 = iota + (g * 16)
                for h in range(8):
                    h16 = jnp.full((16,), h, jnp.int32)
                    num = plsc.load_gather(exb, [e16, h16])
                    dd = plsc.load_gather(den_v, [d16 + h])
                    a = num / (dd + 1e-9)
                    if has_res:
                        a = a * (1.0 - ALPHA) + \
                            plsc.load_gather(rb, [e16, h16]) * ALPHA
                    plsc.store_scatter(exb, [e16, h16], a)
            pltpu.sync_copy(exb, a_h.at[pl.ds(base, CH)])
            return carry

        lax.fori_loop(0, NCH, chunk_a, 0)

    if has_res:
        return k(ex, den_sum, dstp, res)
    return k(ex, den_sum, dstp)


def _sc_gather_scale(a, srcp, featT2):
    @functools.partial(
        pl.kernel,
        mesh=_sc_mesh(),
        compiler_params=pltpu.CompilerParams(needs_layout_passes=False),
        out_type=jax.ShapeDtypeStruct((4, EP, 128), jnp.float32),
        scratch_types=[
            pltpu.VMEM((CHA,), jnp.int32),
            pltpu.VMEM((CHA,), jnp.int32),
            pltpu.VMEM((CHA, 16), jnp.float32),
            pltpu.VMEM((CHA, 128), jnp.float32),
            pltpu.VMEM((CHA, 128), jnp.float32),
        ],
    )
    def k(a_h, src_h, ft_h, msg_h, si_v, gi_v, ab, fbuf, msg):
        cid = lax.axis_index("c")
        tid = lax.axis_index("s")
        wid = tid * 2 + cid

        for hg in range(4):
            def chunk_g(j, carry):
                base = wid * (NCH * CH) + j * CHA
                pltpu.sync_copy(src_h.at[pl.ds(base, CHA)], si_v)
                pltpu.sync_copy(a_h.at[pl.ds(base, CHA)], ab)
                for g in range(CHA // 16):
                    gi_v[pl.ds(g * 16, 16)] = si_v[pl.ds(g * 16, 16)] * 4 + hg
                pltpu.sync_copy(ft_h.at[gi_v], fbuf)
                h0 = jnp.full((16,), 2 * hg, jnp.int32)
                h1 = h0 + 1

                def edge(e, carry2):
                    e16 = jnp.full((16,), e, jnp.int32)
                    av0 = plsc.load_gather(ab, [e16, h0])
                    av1 = plsc.load_gather(ab, [e16, h1])
                    for q in range(4):
                        msg[e, pl.ds(q * 16, 16)] = \
                            fbuf[e, pl.ds(q * 16, 16)] * av0
                        msg[e, pl.ds(64 + q * 16, 16)] = \
                            fbuf[e, pl.ds(64 + q * 16, 16)] * av1
                    return carry2

                lax.fori_loop(0, CHA, edge, 0, unroll=4)
                pltpu.sync_copy(msg, msg_h.at[hg, pl.ds(base, CHA)])
                return carry

            lax.fori_loop(0, NCH, chunk_g, 0)

    return k(a, srcp, featT2)


def _sc_pair_gather(z, left, right):
    CP = 128

    @functools.partial(
        pl.kernel,
        mesh=_sc_mesh(),
        compiler_params=pltpu.CompilerParams(needs_layout_passes=False),
        out_type=(
            jax.ShapeDtypeStruct((P, 256), jnp.float32),
            jax.ShapeDtypeStruct((P, 256), jnp.float32),
        ),
        scratch_types=[
            pltpu.VMEM((CP,), jnp.int32),
            pltpu.VMEM((CP,), jnp.int32),
            pltpu.VMEM((CP, 256), jnp.float32),
            pltpu.VMEM((CP, 256), jnp.float32),
        ],
    )
    def k(z_h, l_h, r_h, le_h, re_h, li_v, ri_v, lrow, rrow):
        cid = lax.axis_index("c")
        tid = lax.axis_index("s")
        wid = tid * 2 + cid

        def step(j, carry):
            base = wid * (P // NW) + j * CP
            pltpu.sync_copy(l_h.at[pl.ds(base, CP)], li_v)
            pltpu.sync_copy(r_h.at[pl.ds(base, CP)], ri_v)
            pltpu.sync_copy(z_h.at[li_v], lrow)
            pltpu.sync_copy(z_h.at[ri_v], rrow)
            pltpu.sync_copy(lrow, le_h.at[pl.ds(base, CP)])
            pltpu.sync_copy(rrow, re_h.at[pl.ds(base, CP)])
            return carry

        lax.fori_loop(0, (P // NW) // CP, step, 0)

    return k(z, left, right)


def _l2n(x):
    return x / jnp.maximum(jnp.linalg.norm(x, axis=1, keepdims=True), 1e-12)


def _attn_proj(al, ar):
    d = al.shape[1]
    blocks = []
    for h in range(H):
        z = jnp.zeros((d, 16), jnp.float32)
        z = z.at[:, h].set(al[h]).at[:, 8 + h].set(ar[h])
        blocks.append(z)
    return jnp.concatenate(blocks, axis=0)


def _gat(h, fcW, eemb, fceW, al, ar, ae, srcp, dstp, efp,
         res_attn, residual, act):
    feat = _mm(h, fcW)
    elr = _mm(feat, _attn_proj(al, ar))
    ea_rel = ((eemb @ fceW).reshape(NE, H, ED) * ae[None]).sum(-1)
    el_flat = elr[:, :8].reshape(NP * 8)
    er_flat = elr[:, 8:].reshape(NP * 8)
    ea_flat = jnp.zeros((8, 8), jnp.float32).at[:NE].set(ea_rel).reshape(64)

    part = _sc_logits_part(el_flat, ea_flat, srcp, efp)
    ex = _sc_logits_ex(er_flat, part, dstp)
    den = jax.ops.segment_sum(ex[:, :8], dstp, num_segments=NP)
    den_sum = den.reshape(NP * 8)
    featT2 = feat.reshape(NP * 4, 128)
    a = _sc_norm(ex, den_sum, dstp, res_attn)
    msg = _sc_gather_scale(a, srcp, featT2)
    accs = jax.ops.segment_sum(
        msg.reshape(4 * EP, 128),
        (jnp.arange(4, dtype=jnp.int32)[:, None] * NP + dstp[None, :]).reshape(-1),
        num_segments=4 * NP).reshape(4, NP, 128)
    rst = jnp.transpose(accs, (1, 0, 2)).reshape(NP, 512)
    if residual:
        rst = rst + h
    if act:
        rst = jax.nn.elu(rst)
    return rst, a


def kernel(feat0, feat1, feat2, fc0_w, fc0_b, fc1_w, fc1_b, fc2_w, fc2_b,
           g0_fc, g0_eemb, g0_fce, g0_al, g0_ar, g0_ae,
           g1_fc, g1_eemb, g1_fce, g1_al, g1_ar, g1_ae,
           g2_fc, g2_eemb, g2_fce, g2_al, g2_ar, g2_ae, dec_W,
           edge_index, e_feat, left, right, mid):
    src, dst = edge_index[0], edge_index[1]
    pad = jnp.full((EP - E,), N, jnp.int32)
    srcp = jnp.concatenate([src.astype(jnp.int32), pad])
    dstp = jnp.concatenate([dst.astype(jnp.int32), pad])
    efp = jnp.concatenate([e_feat.astype(jnp.int32),
                           jnp.zeros((EP - E,), jnp.int32)])

    h0 = jnp.concatenate([
        _mm(jnp.pad(feat0, ((0, 96), (0, 0))), fc0_w)[:4000] + fc0_b,
        _mm(jnp.pad(feat1, ((0, 72), (0, 0))), fc1_w)[:3000] + fc1_b,
        _mm(jnp.pad(feat2, ((0, 72), (0, 0))), fc2_w)[:3000] + fc2_b,
    ], axis=0)
    h0 = jnp.pad(h0, ((0, NP - N), (0, 0)))
    emb0 = _l2n(h0[:N])

    h1, a1 = _gat(h0, g0_fc, g0_eemb, g0_fce, g0_al, g0_ar, g0_ae,
                  srcp, dstp, efp, None, False, True)
    emb1 = _l2n(h1[:N].reshape(N, H, NH).mean(1))
    h2, a2 = _gat(h1, g1_fc, g1_eemb, g1_fce, g1_al, g1_ar, g1_ae,
                  srcp, dstp, efp, a1, True, True)
    emb2 = _l2n(h2[:N].reshape(N, H, NH).mean(1))
    h3, _ = _gat(h2, g2_fc, g2_eemb, g2_fce, g2_al, g2_ar, g2_ae,
                 srcp, dstp, efp, a2, True, False)
    emb3 = _l2n(h3[:N].reshape(N, H, NC).mean(1))

    z = jnp.concatenate([emb0, emb1, emb2, emb3], axis=1)
    if _USE_SC_PAIR:
        le, re = _sc_pair_gather(z, left.astype(jnp.int32),
                                 right.astype(jnp.int32))
    else:
        le, re = z[left], z[right]
    scores = jnp.zeros((P,), jnp.float32)
    for r in range(NE):
        t = (_mm(le, dec_W[r]) * re).sum(1)
        scores = jnp.where(mid == r, t, scores)
    return jax.nn.sigmoid(scores)
